# baseline (device time: 14032 ns/iter reference)
import jax
import jax.numpy as jnp
from jax import lax
from jax.experimental import pallas as pl
from jax.experimental.pallas import tpu as pltpu


def kernel(x, dy, gamma):
    m, d = x.shape
    mq = m // 4

    def body(x_hbm, dy_hbm, out_ref,
             xq_ref, dyq_ref, acc_ref, comm_ref,
             copy_sems, send_sems, recv_sems):
        my_x = lax.axis_index("x")
        my_y = lax.axis_index("y")
        my_z = lax.axis_index("z")
        nbrs = [
            (1 - my_x, my_y, my_z),
            (my_x, 1 - my_y, my_z),
            (my_x, my_y, 1 - my_z),
        ]

        barrier_sem = pltpu.get_barrier_semaphore()
        for nbr in nbrs:
            pl.semaphore_signal(
                barrier_sem, inc=1,
                device_id=nbr, device_id_type=pl.DeviceIdType.MESH,
            )

        q = 2 * my_x + my_z
        r0 = q * mq
        cx = pltpu.make_async_copy(
            x_hbm.at[pl.ds(r0, mq), :], xq_ref, copy_sems.at[0]
        )
        cd = pltpu.make_async_copy(
            dy_hbm.at[pl.ds(r0, mq), :], dyq_ref, copy_sems.at[1]
        )
        cx.start()
        cd.start()
        cx.wait()
        cd.wait()

        xv = xq_ref[:, :]
        dyv = dyq_ref[:, :]
        mu = jnp.mean(xv, axis=1, keepdims=True)
        msq = jnp.mean(xv * xv, axis=1, keepdims=True)
        rstd = lax.rsqrt(msq - mu * mu + 1e-5)
        xhat = (xv - mu) * rstd
        acc_ref[0, :] = jnp.sum(dyv * xhat, axis=0)
        acc_ref[1, :] = jnp.sum(dyv, axis=0)

        pl.semaphore_wait(barrier_sem, 3)

        for k, nbr in enumerate(nbrs):
            rdma = pltpu.make_async_remote_copy(
                src_ref=acc_ref,
                dst_ref=comm_ref.at[k],
                send_sem=send_sems.at[k],
                recv_sem=recv_sems.at[k],
                device_id=nbr,
                device_id_type=pl.DeviceIdType.MESH,
            )
            rdma.start()
            rdma.wait()
            acc_ref[:, :] = acc_ref[:, :] + comm_ref[k, :, :]

        out_ref[:, :] = acc_ref[:, :]

    return pl.pallas_call(
        body,
        out_shape=jax.ShapeDtypeStruct((2, d), jnp.float32),
        in_specs=[
            pl.BlockSpec(memory_space=pl.ANY),
            pl.BlockSpec(memory_space=pl.ANY),
        ],
        out_specs=pl.BlockSpec(memory_space=pltpu.VMEM),
        scratch_shapes=[
            pltpu.VMEM((mq, d), jnp.float32),
            pltpu.VMEM((mq, d), jnp.float32),
            pltpu.VMEM((2, d), jnp.float32),
            pltpu.VMEM((3, 2, d), jnp.float32),
            pltpu.SemaphoreType.DMA((2,)),
            pltpu.SemaphoreType.DMA((3,)),
            pltpu.SemaphoreType.DMA((3,)),
        ],
        compiler_params=pltpu.CompilerParams(collective_id=0),
    )(x, dy)


# device time: 11729 ns/iter; 1.1964x vs baseline; 1.1964x over previous
import jax
import jax.numpy as jnp
from jax import lax
from jax.experimental import pallas as pl
from jax.experimental.pallas import tpu as pltpu

N_DEV = 8


def kernel(x, dy, gamma):
    m, d = x.shape
    mq = m // 4

    def body(x_hbm, dy_hbm, out_ref,
             xq_ref, dyq_ref, partial_ref, comm_ref,
             copy_sems, send_sems, recv_sems):
        my_x = lax.axis_index("x")
        my_y = lax.axis_index("y")
        my_z = lax.axis_index("z")
        my_flat = my_x * 4 + my_y * 2 + my_z

        barrier_sem = pltpu.get_barrier_semaphore()
        for p in range(N_DEV):
            pxyz = (p >> 2, (p >> 1) & 1, p & 1)

            @pl.when(p != my_flat)
            def _():
                pl.semaphore_signal(
                    barrier_sem, inc=1,
                    device_id=pxyz, device_id_type=pl.DeviceIdType.MESH,
                )

        q = 2 * my_x + my_z
        r0 = q * mq
        cx = pltpu.make_async_copy(
            x_hbm.at[pl.ds(r0, mq), :], xq_ref, copy_sems.at[0]
        )
        cd = pltpu.make_async_copy(
            dy_hbm.at[pl.ds(r0, mq), :], dyq_ref, copy_sems.at[1]
        )
        cx.start()
        cd.start()
        cx.wait()
        cd.wait()

        xv = xq_ref[:, :]
        dyv = dyq_ref[:, :]
        mu = jnp.mean(xv, axis=1, keepdims=True)
        msq = jnp.mean(xv * xv, axis=1, keepdims=True)
        rstd = lax.rsqrt(msq - mu * mu + 1e-5)
        xhat = (xv - mu) * rstd
        partial_ref[0, :] = jnp.sum(dyv * xhat, axis=0)
        partial_ref[1, :] = jnp.sum(dyv, axis=0)

        pl.semaphore_wait(barrier_sem, N_DEV - 1)

        rdmas = []
        for p in range(N_DEV):
            pxyz = (p >> 2, (p >> 1) & 1, p & 1)
            rdma = pltpu.make_async_remote_copy(
                src_ref=partial_ref,
                dst_ref=comm_ref.at[my_flat],
                send_sem=send_sems.at[p],
                recv_sem=recv_sems.at[my_flat],
                device_id=pxyz,
                device_id_type=pl.DeviceIdType.MESH,
            )
            rdmas.append(rdma)

            @pl.when(p != my_flat)
            def _():
                rdma.start()

        comm_ref[pl.ds(my_flat, 1), :, :] = partial_ref[:, :][None]

        for p in range(N_DEV):
            recv = pltpu.make_async_remote_copy(
                src_ref=partial_ref,
                dst_ref=comm_ref.at[p],
                send_sem=send_sems.at[p],
                recv_sem=recv_sems.at[p],
                device_id=(0, 0, 0),
                device_id_type=pl.DeviceIdType.MESH,
            )

            @pl.when(p != my_flat)
            def _():
                recv.wait_recv()
                rdmas[p].wait_send()

        out_ref[:, :] = jnp.sum(comm_ref[:, :, :], axis=0)

    return pl.pallas_call(
        body,
        out_shape=jax.ShapeDtypeStruct((2, d), jnp.float32),
        in_specs=[
            pl.BlockSpec(memory_space=pl.ANY),
            pl.BlockSpec(memory_space=pl.ANY),
        ],
        out_specs=pl.BlockSpec(memory_space=pltpu.VMEM),
        scratch_shapes=[
            pltpu.VMEM((mq, d), jnp.float32),
            pltpu.VMEM((mq, d), jnp.float32),
            pltpu.VMEM((2, d), jnp.float32),
            pltpu.VMEM((N_DEV, 2, d), jnp.float32),
            pltpu.SemaphoreType.DMA((2,)),
            pltpu.SemaphoreType.DMA((N_DEV,)),
            pltpu.SemaphoreType.DMA((N_DEV,)),
        ],
        compiler_params=pltpu.CompilerParams(collective_id=0),
    )(x, dy)


# device time: 11696 ns/iter; 1.1997x vs baseline; 1.0028x over previous
import jax
import jax.numpy as jnp
from jax import lax
from jax.experimental import pallas as pl
from jax.experimental.pallas import tpu as pltpu

N_DEV = 8


def kernel(x, dy, gamma):
    m, d = x.shape
    mq = m // 4

    def body(x_hbm, dy_hbm, out_ref,
             xq_ref, dyq_ref, partial_ref, comm_ref,
             copy_sems, send_sems, recv_sems):
        my_x = lax.axis_index("x")
        my_y = lax.axis_index("y")
        my_z = lax.axis_index("z")
        my_flat = my_x * 4 + my_y * 2 + my_z

        barrier_sem = pltpu.get_barrier_semaphore()
        for p in range(N_DEV):
            pxyz = (p >> 2, (p >> 1) & 1, p & 1)

            @pl.when(p != my_flat)
            def _():
                pl.semaphore_signal(
                    barrier_sem, inc=1,
                    device_id=pxyz, device_id_type=pl.DeviceIdType.MESH,
                )

        q = 2 * my_x + my_z
        r0 = q * mq
        cx = pltpu.make_async_copy(
            x_hbm.at[pl.ds(r0, mq), :], xq_ref, copy_sems.at[0]
        )
        cd = pltpu.make_async_copy(
            dy_hbm.at[pl.ds(r0, mq), :], dyq_ref, copy_sems.at[1]
        )
        cx.start()
        cd.start()
        cx.wait()
        cd.wait()

        xv = xq_ref[:, :]
        dyv = dyq_ref[:, :]
        mu = jnp.mean(xv, axis=1, keepdims=True)
        msq = jnp.mean(xv * xv, axis=1, keepdims=True)
        rstd = lax.rsqrt(msq - mu * mu + 1e-5)
        xhat = (xv - mu) * rstd
        partial_ref[0, :] = jnp.sum(dyv * xhat, axis=0)
        partial_ref[1, :] = jnp.sum(dyv, axis=0)

        pl.semaphore_wait(barrier_sem, N_DEV - 1)

        rdmas = []
        for p in range(N_DEV):
            pxyz = (p >> 2, (p >> 1) & 1, p & 1)
            rdma = pltpu.make_async_remote_copy(
                src_ref=partial_ref,
                dst_ref=comm_ref.at[my_flat],
                send_sem=send_sems.at[p],
                recv_sem=recv_sems.at[my_flat],
                device_id=pxyz,
                device_id_type=pl.DeviceIdType.MESH,
            )
            rdmas.append(rdma)

            @pl.when(p != my_flat)
            def _():
                rdma.start()

        comm_ref[pl.ds(my_flat, 1), :, :] = partial_ref[:, :][None]

        for p in range(N_DEV):
            recv = pltpu.make_async_remote_copy(
                src_ref=partial_ref,
                dst_ref=comm_ref.at[p],
                send_sem=send_sems.at[p],
                recv_sem=recv_sems.at[p],
                device_id=(0, 0, 0),
                device_id_type=pl.DeviceIdType.MESH,
            )

            @pl.when(p != my_flat)
            def _():
                recv.wait_recv()
                rdmas[p].wait_send()

        out_ref[:, :] = jnp.sum(comm_ref[:, :, :], axis=0)

    return pl.pallas_call(
        body,
        out_shape=jax.ShapeDtypeStruct((2, d), jnp.float32),
        in_specs=[
            pl.BlockSpec(memory_space=pltpu.MemorySpace.HBM),
            pl.BlockSpec(memory_space=pltpu.MemorySpace.HBM),
        ],
        out_specs=pl.BlockSpec(memory_space=pltpu.VMEM),
        scratch_shapes=[
            pltpu.VMEM((mq, d), jnp.float32),
            pltpu.VMEM((mq, d), jnp.float32),
            pltpu.VMEM((2, d), jnp.float32),
            pltpu.VMEM((N_DEV, 2, d), jnp.float32),
            pltpu.SemaphoreType.DMA((2,)),
            pltpu.SemaphoreType.DMA((N_DEV,)),
            pltpu.SemaphoreType.DMA((N_DEV,)),
        ],
        compiler_params=pltpu.CompilerParams(collective_id=0),
    )(x, dy)


# device time: 9217 ns/iter; 1.5224x vs baseline; 1.2690x over previous
import jax
import jax.numpy as jnp
from jax import lax
from jax.experimental import pallas as pl
from jax.experimental.pallas import tpu as pltpu

N_DEV = 8


def kernel(x, dy, gamma):
    m, d = x.shape
    mq = m // 4

    my_x = lax.axis_index("x")
    my_z = lax.axis_index("z")
    r0 = (2 * my_x + my_z) * mq
    xq = lax.dynamic_slice(x, (r0, 0), (mq, d))
    dyq = lax.dynamic_slice(dy, (r0, 0), (mq, d))

    def body(xq_ref, dyq_ref, out_ref,
             partial_ref, comm_ref, send_sems, recv_sems):
        my_flat = (
            lax.axis_index("x") * 4
            + lax.axis_index("y") * 2
            + lax.axis_index("z")
        )

        barrier_sem = pltpu.get_barrier_semaphore()
        for p in range(N_DEV):
            pxyz = (p >> 2, (p >> 1) & 1, p & 1)

            @pl.when(p != my_flat)
            def _():
                pl.semaphore_signal(
                    barrier_sem, inc=1,
                    device_id=pxyz, device_id_type=pl.DeviceIdType.MESH,
                )

        xv = xq_ref[:, :]
        dyv = dyq_ref[:, :]
        mu = jnp.mean(xv, axis=1, keepdims=True)
        msq = jnp.mean(xv * xv, axis=1, keepdims=True)
        rstd = lax.rsqrt(msq - mu * mu + 1e-5)
        xhat = (xv - mu) * rstd
        partial_ref[0, :] = jnp.sum(dyv * xhat, axis=0)
        partial_ref[1, :] = jnp.sum(dyv, axis=0)

        pl.semaphore_wait(barrier_sem, N_DEV - 1)

        rdmas = []
        for p in range(N_DEV):
            pxyz = (p >> 2, (p >> 1) & 1, p & 1)
            rdma = pltpu.make_async_remote_copy(
                src_ref=partial_ref,
                dst_ref=comm_ref.at[my_flat],
                send_sem=send_sems.at[p],
                recv_sem=recv_sems.at[my_flat],
                device_id=pxyz,
                device_id_type=pl.DeviceIdType.MESH,
            )
            rdmas.append(rdma)

            @pl.when(p != my_flat)
            def _():
                rdma.start()

        comm_ref[pl.ds(my_flat, 1), :, :] = partial_ref[:, :][None]

        for p in range(N_DEV):
            recv = pltpu.make_async_remote_copy(
                src_ref=partial_ref,
                dst_ref=comm_ref.at[p],
                send_sem=send_sems.at[p],
                recv_sem=recv_sems.at[p],
                device_id=(0, 0, 0),
                device_id_type=pl.DeviceIdType.MESH,
            )

            @pl.when(p != my_flat)
            def _():
                recv.wait_recv()
                rdmas[p].wait_send()

        out_ref[:, :] = jnp.sum(comm_ref[:, :, :], axis=0)

    return pl.pallas_call(
        body,
        out_shape=jax.ShapeDtypeStruct((2, d), jnp.float32),
        in_specs=[
            pl.BlockSpec(memory_space=pltpu.VMEM),
            pl.BlockSpec(memory_space=pltpu.VMEM),
        ],
        out_specs=pl.BlockSpec(memory_space=pltpu.VMEM),
        scratch_shapes=[
            pltpu.VMEM((2, d), jnp.float32),
            pltpu.VMEM((N_DEV, 2, d), jnp.float32),
            pltpu.SemaphoreType.DMA((N_DEV,)),
            pltpu.SemaphoreType.DMA((N_DEV,)),
        ],
        compiler_params=pltpu.CompilerParams(collective_id=0),
    )(xq, dyq)
